# baseline (speedup 1.0000x reference)
import functools

import jax
import jax.numpy as jnp
from jax import lax
from jax.experimental import pallas as pl
from jax.experimental.pallas import tpu as pltpu
from jax.experimental.pallas import tpu_sc as plsc

NC = 2
NS = 16
NW = NC * NS

EPS = 1e-3
CHUNK = 128
HJ = 8


def _rsqrt_newton(v):
    i = lax.bitcast_convert_type(v, jnp.int32)
    i = jnp.int32(0x5F3759DF) - lax.shift_right_logical(i, 1)
    y = lax.bitcast_convert_type(i, jnp.float32)
    for _ in range(3):
        y = y * (1.5 - 0.5 * v * y * y)
    return y


_GDN = lax.GatherDimensionNumbers(
    offset_dims=(), collapsed_slice_dims=(0,), start_index_map=(0,))


def _shuffle(x, idx):
    return lax.gather(x, idx[:, None], _GDN, (1,),
                      mode=lax.GatherScatterMode.PROMISE_IN_BOUNDS)


def _lane_sum(x, iota):
    for k in (8, 4, 2, 1):
        x = x + _shuffle(x, iota ^ k)
    return x


def _sc_fused(table, ids_flat, aug_idx_flat, type_pad, pos200, gamma, beta):
    n_rows = ids_flat.shape[0]
    H = table.shape[1]
    S = pos200.shape[0]
    per_w = n_rows // NW
    chunks_per_w = per_w // CHUNK
    mesh = plsc.VectorSubcoreMesh(core_axis_name="c", subcore_axis_name="s")

    @functools.partial(
        pl.kernel,
        out_type=jax.ShapeDtypeStruct((n_rows, H), jnp.float32),
        mesh=mesh,
        scratch_types=[
            pltpu.VMEM((2 * S, H), jnp.float32),
            pltpu.VMEM((CHUNK, H), jnp.float32),
            pltpu.VMEM((CHUNK, H), jnp.float32),
            pltpu.VMEM((CHUNK, H), jnp.float32),
            pltpu.VMEM((CHUNK, H), jnp.float32),
            pltpu.VMEM((per_w,), jnp.int32),
            pltpu.VMEM((per_w,), jnp.int32),
            pltpu.VMEM((8, H), jnp.float32),
            pltpu.VMEM((H,), jnp.float32),
            pltpu.VMEM((H,), jnp.float32),
            pltpu.SemaphoreType.DMA,
            pltpu.SemaphoreType.DMA,
            pltpu.SemaphoreType.DMA,
            pltpu.SemaphoreType.DMA,
        ],
    )
    def k(table_hbm, ids_hbm, aidx_hbm, type_hbm, pos_hbm, gamma_hbm,
          beta_hbm, out_hbm, aug, rows0, rows1, out0, out1, idxv, aidxv,
          typev, gv, bv, g0, g1, s0, s1):
        wid = lax.axis_index("s") * NC + lax.axis_index("c")
        base = wid * per_w

        pltpu.sync_copy(ids_hbm.at[pl.ds(base, per_w)], idxv)
        pltpu.sync_copy(aidx_hbm.at[pl.ds(base, per_w)], aidxv)
        pltpu.sync_copy(pos_hbm, aug.at[pl.ds(0, S)])
        pltpu.sync_copy(pos_hbm, aug.at[pl.ds(S, S)])
        pltpu.sync_copy(type_hbm, typev)
        pltpu.sync_copy(gamma_hbm, gv)
        pltpu.sync_copy(beta_hbm, bv)

        def aug_body(p, carry):
            for j in range(HJ):
                sl = pl.ds(16 * j, 16)
                aug[p, sl] += typev[0, sl]
                aug[S + p, sl] += typev[1, sl]
            return carry
        lax.fori_loop(0, S, aug_body, 0)

        def fire_gather(c, buf, sem):
            return pltpu.async_copy(
                table_hbm.at[idxv.at[pl.ds(c * CHUNK, CHUNK)]], buf, sem)

        def wait_gather(c, buf, sem):
            pltpu.make_async_copy(
                table_hbm.at[idxv.at[pl.ds(c * CHUNK, CHUNK)]], buf,
                sem).wait()

        def fire_store(c, buf, sem):
            return pltpu.async_copy(
                buf, out_hbm.at[pl.ds((base + c * CHUNK), CHUNK)], sem)

        def wait_store(c, buf, sem):
            pltpu.make_async_copy(
                buf, out_hbm.at[pl.ds((base + c * CHUNK), CHUNK)],
                sem).wait()

        def compute(c, rows, outb):
            gs = [gv[pl.ds(16 * j, 16)] for j in range(HJ)]
            bs = [bv[pl.ds(16 * j, 16)] for j in range(HJ)]

            iota = lax.iota(jnp.int32, 16)

            def gb(g, carry):
                arv = aidxv[pl.ds(c * CHUNK + 16 * g, 16)]
                for t16 in range(16):
                    t = g * 16 + t16
                    ar = arv[t16]
                    ys = []
                    for j in range(HJ):
                        sl = pl.ds(16 * j, 16)
                        ys.append(rows[t, sl] + aug[ar, sl])
                    sv = ((ys[0] + ys[1]) + (ys[2] + ys[3])) + \
                         ((ys[4] + ys[5]) + (ys[6] + ys[7]))
                    sq = [y * y for y in ys]
                    qv = ((sq[0] + sq[1]) + (sq[2] + sq[3])) + \
                         ((sq[4] + sq[5]) + (sq[6] + sq[7]))
                    mean = _lane_sum(sv, iota) * (1.0 / H)
                    var = _lane_sum(qv, iota) * (1.0 / H) - mean * mean
                    rstd = _rsqrt_newton(var + EPS)
                    shift = -mean * rstd
                    for j in range(HJ):
                        sl = pl.ds(16 * j, 16)
                        outb[t, sl] = (ys[j] * rstd + shift) * gs[j] + bs[j]
                return carry

            lax.fori_loop(0, CHUNK // 16, gb, 0)

        fire_gather(0, rows0, g0)

        def step(kk, carry):
            c0 = 2 * kk
            c1 = c0 + 1
            fire_gather(c1, rows1, g1)
            wait_gather(c0, rows0, g0)

            @pl.when(kk > 0)
            def _():
                wait_store(c0 - 2, out0, s0)
            compute(c0, rows0, out0)
            fire_store(c0, out0, s0)

            @pl.when(kk < chunks_per_w // 2 - 1)
            def _():
                fire_gather(c0 + 2, rows0, g0)
            wait_gather(c1, rows1, g1)

            @pl.when(kk > 0)
            def _():
                wait_store(c1 - 2, out1, s1)
            compute(c1, rows1, out1)
            fire_store(c1, out1, s1)
            return carry

        lax.fori_loop(0, chunks_per_w // 2, step, 0)
        wait_store(chunks_per_w - 2, out0, s0)
        wait_store(chunks_per_w - 1, out1, s1)

    return k(table, ids_flat, aug_idx_flat, type_pad, pos200, gamma, beta)


def kernel(input_ids, token_type_ids, word_emb, type_emb, pos_emb, gamma, beta):
    B, S = input_ids.shape
    H = word_emb.shape[1]
    n_rows = B * S
    ids_flat = input_ids.reshape(n_rows)
    pos_row = lax.broadcasted_iota(jnp.int32, (B, S), 1)
    aug_idx = (token_type_ids * S + pos_row).reshape(n_rows)
    type_pad = jnp.pad(type_emb, ((0, 6), (0, 0)))
    out = _sc_fused(word_emb, ids_flat, aug_idx, type_pad, pos_emb[:S],
                    gamma, beta)
    return out.reshape(B, S, H)


# speedup vs baseline: 1.0159x; 1.0159x over previous
import functools

import jax
import jax.numpy as jnp
from jax import lax
from jax.experimental import pallas as pl
from jax.experimental.pallas import tpu as pltpu
from jax.experimental.pallas import tpu_sc as plsc

NC = 2
NS = 16
NW = NC * NS

EPS = 1e-3
CHUNK = 128
HJ = 8

_GDN = lax.GatherDimensionNumbers(
    offset_dims=(), collapsed_slice_dims=(0,), start_index_map=(0,))


def _shuffle(x, idx):
    return lax.gather(x, idx[:, None], _GDN, (1,),
                      mode=lax.GatherScatterMode.PROMISE_IN_BOUNDS)


def _lane_sum(x, iota):
    for k in (8, 4, 2, 1):
        x = x + _shuffle(x, iota ^ k)
    return x


def _rsqrt_newton(v):
    i = lax.bitcast_convert_type(v, jnp.int32)
    i = jnp.int32(0x5F3759DF) - lax.shift_right_logical(i, 1)
    y = lax.bitcast_convert_type(i, jnp.float32)
    for _ in range(3):
        y = y * (1.5 - 0.5 * v * y * y)
    return y


def _sc_fused(table, ids_flat, tt_flat, type_pad, pos200, gamma, beta):
    n_rows = ids_flat.shape[0]
    H = table.shape[1]
    S = pos200.shape[0]
    per_w = n_rows // NW
    chunks_per_w = per_w // CHUNK
    mesh = plsc.VectorSubcoreMesh(core_axis_name="c", subcore_axis_name="s")

    @functools.partial(
        pl.kernel,
        out_type=jax.ShapeDtypeStruct((n_rows, H), jnp.float32),
        mesh=mesh,
        scratch_types=[
            pltpu.VMEM((S, H), jnp.float32),
            pltpu.VMEM((per_w,), jnp.int32),
            pltpu.VMEM((CHUNK, H), jnp.float32),
            pltpu.VMEM((CHUNK, H), jnp.float32),
            pltpu.VMEM((CHUNK, H), jnp.float32),
            pltpu.VMEM((CHUNK, H), jnp.float32),
            pltpu.VMEM((per_w,), jnp.int32),
            pltpu.VMEM((8, H), jnp.float32),
            pltpu.VMEM((H,), jnp.float32),
            pltpu.VMEM((H,), jnp.float32),
            pltpu.SemaphoreType.DMA,
            pltpu.SemaphoreType.DMA,
            pltpu.SemaphoreType.DMA,
            pltpu.SemaphoreType.DMA,
        ],
    )
    def k(table_hbm, ids_hbm, tt_hbm, type_hbm, pos_hbm, gamma_hbm,
          beta_hbm, out_hbm, aug, ttv, rows0, rows1, out0, out1, idxv,
          typev, gv, bv, g0, g1, s0, s1):
        wid = lax.axis_index("s") * NC + lax.axis_index("c")
        base = wid * per_w

        pltpu.sync_copy(ids_hbm.at[pl.ds(base, per_w)], idxv)
        pltpu.sync_copy(tt_hbm.at[pl.ds(base, per_w)], ttv)
        pltpu.sync_copy(pos_hbm, aug)
        pltpu.sync_copy(type_hbm, typev)
        pltpu.sync_copy(gamma_hbm, gv)
        pltpu.sync_copy(beta_hbm, bv)

        def aug_body(p, carry):
            for j in range(HJ):
                sl = pl.ds(16 * j, 16)
                aug[p, sl] += typev[0, sl]
            return carry
        lax.fori_loop(0, S, aug_body, 0)

        def fire_gather(c, buf, sem):
            return pltpu.async_copy(
                table_hbm.at[idxv.at[pl.ds(c * CHUNK, CHUNK)]], buf, sem)

        def wait_gather(c, buf, sem):
            pltpu.make_async_copy(
                table_hbm.at[idxv.at[pl.ds(c * CHUNK, CHUNK)]], buf,
                sem).wait()

        def fire_store(c, buf, sem):
            return pltpu.async_copy(
                buf, out_hbm.at[pl.ds((base + c * CHUNK), CHUNK)], sem)

        def wait_store(c, buf, sem):
            pltpu.make_async_copy(
                buf, out_hbm.at[pl.ds((base + c * CHUNK), CHUNK)],
                sem).wait()

        iota = lax.iota(jnp.int32, 16)

        def compute(c, rows, outb):
            gs = tuple(gv[pl.ds(16 * j, 16)] for j in range(HJ))
            bs = tuple(bv[pl.ds(16 * j, 16)] for j in range(HJ))
            dv = tuple(typev[1, pl.ds(16 * j, 16)] -
                       typev[0, pl.ds(16 * j, 16)] for j in range(HJ))

            def gb(g, carry):
                gs, bs, dv = carry
                goff = c * CHUNK + 16 * g
                ttgf = ttv[pl.ds(goff, 16)].astype(jnp.float32)
                pg = lax.rem(base + goff, S)
                for t16 in range(16):
                    t = g * 16 + t16
                    pr = pg + t16
                    pr = jnp.where(pr >= S, pr - S, pr)
                    ttf = _shuffle(ttgf, (iota & 0) + t16)
                    ys = []
                    for j in range(HJ):
                        sl = pl.ds(16 * j, 16)
                        ys.append(rows[t, sl] + aug[pr, sl] + ttf * dv[j])
                    sv = ((ys[0] + ys[1]) + (ys[2] + ys[3])) + \
                         ((ys[4] + ys[5]) + (ys[6] + ys[7]))
                    sq = [y * y for y in ys]
                    qv = ((sq[0] + sq[1]) + (sq[2] + sq[3])) + \
                         ((sq[4] + sq[5]) + (sq[6] + sq[7]))
                    mean = _lane_sum(sv, iota) * (1.0 / H)
                    var = _lane_sum(qv, iota) * (1.0 / H) - mean * mean
                    rstd = _rsqrt_newton(var + EPS)
                    shift = -mean * rstd
                    for j in range(HJ):
                        sl = pl.ds(16 * j, 16)
                        outb[t, sl] = (ys[j] * rstd + shift) * gs[j] + bs[j]
                return carry

            lax.fori_loop(0, CHUNK // 16, gb, (gs, bs, dv))

        fire_gather(0, rows0, g0)

        def step(kk, carry):
            c0 = 2 * kk
            c1 = c0 + 1
            fire_gather(c1, rows1, g1)
            wait_gather(c0, rows0, g0)

            @pl.when(kk > 0)
            def _():
                wait_store(c0 - 2, out0, s0)
            compute(c0, rows0, out0)
            fire_store(c0, out0, s0)

            @pl.when(kk < chunks_per_w // 2 - 1)
            def _():
                fire_gather(c0 + 2, rows0, g0)
            wait_gather(c1, rows1, g1)

            @pl.when(kk > 0)
            def _():
                wait_store(c1 - 2, out1, s1)
            compute(c1, rows1, out1)
            fire_store(c1, out1, s1)
            return carry

        lax.fori_loop(0, chunks_per_w // 2, step, 0)
        wait_store(chunks_per_w - 2, out0, s0)
        wait_store(chunks_per_w - 1, out1, s1)

    return k(table, ids_flat, tt_flat, type_pad, pos200, gamma, beta)


def kernel(input_ids, token_type_ids, word_emb, type_emb, pos_emb, gamma, beta):
    B, S = input_ids.shape
    H = word_emb.shape[1]
    n_rows = B * S
    ids_flat = input_ids.reshape(n_rows)
    tt_flat = token_type_ids.reshape(n_rows)
    type_pad = jnp.pad(type_emb, ((0, 6), (0, 0)))
    out = _sc_fused(word_emb, ids_flat, tt_flat, type_pad, pos_emb[:S],
                    gamma, beta)
    return out.reshape(B, S, H)
